# baseline (speedup 1.0000x reference)
import jax
import jax.numpy as jnp
from jax import lax
from jax.experimental import pallas as pl
from jax.experimental.pallas import tpu as pltpu


def _colstats(m):
    rn = jnp.sqrt(jnp.sum(m * m, axis=1, keepdims=True))
    s = jnp.sum(m / (rn + 1e-8), axis=0, keepdims=True)
    sm = jnp.sum(m, axis=0, keepdims=True)
    sq = jnp.sum(m * m, axis=0, keepdims=True)
    return jnp.concatenate([s, sm, sq], axis=0)


def _sim_var(stats, n):
    s = stats[0, :]
    sim = (jnp.sum(s * s) - n) / (n * (n - 1.0))
    cnt = n * stats.shape[1]
    mean = jnp.sum(stats[1, :]) / cnt
    var = jnp.sum(stats[2, :]) / cnt - mean * mean
    return sim, var


def _gcn_body(adj_ref, x_ref, w1_ref, b1_ref, w2_ref, b2_ref,
              out_ref, m_ref, u_ref, v_ref, xst_ref, hst_ref, ost_ref):
    i = pl.program_id(0)
    ng = pl.num_programs(0)
    g = ng // 2
    tm = adj_ref.shape[0]

    @pl.when(i == 0)
    def _prep():
        x = x_ref[...]
        u = jnp.dot(x, w1_ref[...], preferred_element_type=jnp.float32)
        u_ref[...] = u.astype(jnp.bfloat16)
        xst_ref[...] = _colstats(x)

    a = adj_ref[...].astype(jnp.bfloat16)

    @pl.when(i < g)
    def _layer1():
        h1 = jnp.dot(a, u_ref[...], preferred_element_type=jnp.float32)
        h1 = h1 + b1_ref[...]
        st = _colstats(h1)

        @pl.when(i == 0)
        def _init():
            hst_ref[...] = st

        @pl.when(i > 0)
        def _acc():
            hst_ref[...] += st

        h = jnp.maximum(h1, 0.0).astype(jnp.bfloat16)
        v_ref[pl.ds(i * tm, tm), :] = jnp.dot(
            h, w2_ref[...].astype(jnp.bfloat16),
            preferred_element_type=jnp.float32).astype(jnp.bfloat16)

    @pl.when(i >= g)
    def _layer2():
        o = jnp.dot(a, v_ref[...], preferred_element_type=jnp.float32)
        o = o + b2_ref[...]
        out_ref[...] = o
        st = _colstats(o)

        @pl.when(i == g)
        def _init():
            ost_ref[...] = st

        @pl.when(i > g)
        def _acc():
            ost_ref[...] += st

        @pl.when(i == ng - 1)
        def _finalize():
            n = jnp.float32(tm) * g
            sim1, var1 = _sim_var(xst_ref[...], n)
            sim2, var2 = _sim_var(hst_ref[...], n)
            sim4, var4 = _sim_var(ost_ref[...], n)
            lane = lax.broadcasted_iota(jnp.int32, (1, 16), 1)
            mv = jnp.zeros((1, 16), jnp.float32)
            for k, val in ((0, sim1), (2, var1), (3, sim2), (5, var2),
                           (6, sim2), (8, var2), (9, sim4), (11, var4)):
                mv = jnp.where(lane == k, val, mv)
            m_ref[...] = mv


def kernel(x, adj, W1, b1, W2, b2):
    n, nfeat = x.shape
    nhid = W1.shape[1]
    nclass = W2.shape[1]
    tm = 200
    g = n // tm

    out, mv = pl.pallas_call(
        _gcn_body,
        grid=(2 * g,),
        in_specs=[
            pl.BlockSpec((tm, n), lambda i: (jnp.where(i < g, i, i - g), 0)),
            pl.BlockSpec((n, nfeat), lambda i: (0, 0)),
            pl.BlockSpec((nfeat, nhid), lambda i: (0, 0)),
            pl.BlockSpec((1, nhid), lambda i: (0, 0)),
            pl.BlockSpec((nhid, nclass), lambda i: (0, 0)),
            pl.BlockSpec((1, nclass), lambda i: (0, 0)),
        ],
        out_specs=[
            pl.BlockSpec((tm, nclass),
                         lambda i: (jnp.where(i < g, 0, i - g), 0)),
            pl.BlockSpec((1, 16), lambda i: (0, 0)),
        ],
        out_shape=[
            jax.ShapeDtypeStruct((n, nclass), jnp.float32),
            jax.ShapeDtypeStruct((1, 16), jnp.float32),
        ],
        scratch_shapes=[
            pltpu.VMEM((n, nhid), jnp.bfloat16),
            pltpu.VMEM((n, nclass), jnp.bfloat16),
            pltpu.VMEM((3, nfeat), jnp.float32),
            pltpu.VMEM((3, nhid), jnp.float32),
            pltpu.VMEM((3, nclass), jnp.float32),
        ],
        compiler_params=pltpu.CompilerParams(
            dimension_semantics=("arbitrary",)),
    )(adj, x, W1, b1.reshape(1, nhid), W2, b2.reshape(1, nclass))

    return (out, mv[0, :12])


# speedup vs baseline: 1.0639x; 1.0639x over previous
import jax
import jax.numpy as jnp
from jax import lax
from jax.experimental import pallas as pl
from jax.experimental.pallas import tpu as pltpu


def _colstats(m):
    rn = jnp.sqrt(jnp.sum(m * m, axis=1, keepdims=True))
    s = jnp.sum(m / (rn + 1e-8), axis=0, keepdims=True)
    sm = jnp.sum(m, axis=0, keepdims=True)
    sq = jnp.sum(m * m, axis=0, keepdims=True)
    return jnp.concatenate([s, sm, sq], axis=0)


def _sim_var(stats, n):
    s = stats[0, :]
    sim = (jnp.sum(s * s) - n) / (n * (n - 1.0))
    cnt = n * stats.shape[1]
    mean = jnp.sum(stats[1, :]) / cnt
    var = jnp.sum(stats[2, :]) / cnt - mean * mean
    return sim, var


def _layer1_body(adj_ref, x_ref, w1_ref, b1_ref, w2_ref,
                 v_ref, xst_ref, hst_ref, u_ref):
    i = pl.program_id(0)

    @pl.when(i == 0)
    def _prep():
        x = x_ref[...]
        u = jnp.dot(x, w1_ref[...], preferred_element_type=jnp.float32)
        u_ref[...] = u.astype(jnp.bfloat16)
        xst_ref[0] = _colstats(x)

    a = adj_ref[...].astype(jnp.bfloat16)
    h1 = jnp.dot(a, u_ref[...], preferred_element_type=jnp.float32)
    h1 = h1 + b1_ref[...]
    st = _colstats(h1)

    @pl.when(i == 0)
    def _init():
        hst_ref[0] = st

    @pl.when(i > 0)
    def _acc():
        hst_ref[0] += st

    h = jnp.maximum(h1, 0.0).astype(jnp.bfloat16)
    v_ref[...] = jnp.dot(h, w2_ref[...].astype(jnp.bfloat16),
                         preferred_element_type=jnp.float32
                         ).astype(jnp.bfloat16)


def _layer2_body(adj_ref, v_ref, b2_ref, xst_ref, hst_ref,
                 out_ref, m_ref, acc_ref):
    i = pl.program_id(0)
    ng = pl.num_programs(0)
    a = adj_ref[...].astype(jnp.bfloat16)
    o = jnp.dot(a, v_ref[...], preferred_element_type=jnp.float32)
    o = o + b2_ref[...]
    out_ref[...] = o
    st = _colstats(o)

    @pl.when(i == 0)
    def _init():
        acc_ref[...] = st

    @pl.when(i > 0)
    def _acc():
        acc_ref[...] += st

    @pl.when(i == ng - 1)
    def _finalize():
        n = jnp.float32(out_ref.shape[0]) * ng
        sim1, var1 = _sim_var(xst_ref[0], n)
        sim2, var2 = _sim_var(hst_ref[0], n)
        sim4, var4 = _sim_var(acc_ref[...], n)
        lane = lax.broadcasted_iota(jnp.int32, (1, 12), 1)
        mv = jnp.zeros((1, 12), jnp.float32)
        for k, val in ((0, sim1), (2, var1), (3, sim2), (5, var2),
                       (6, sim2), (8, var2), (9, sim4), (11, var4)):
            mv = jnp.where(lane == k, val, mv)
        m_ref[...] = mv[0]


def kernel(x, adj, W1, b1, W2, b2):
    n, nfeat = x.shape
    nhid = W1.shape[1]
    nclass = W2.shape[1]
    fdt = jnp.float32
    tm = 200
    g = n // tm

    v, xst, hst = pl.pallas_call(
        _layer1_body,
        grid=(g,),
        in_specs=[
            pl.BlockSpec((tm, n), lambda i: (i, 0)),
            pl.BlockSpec((n, nfeat), lambda i: (0, 0)),
            pl.BlockSpec((nfeat, nhid), lambda i: (0, 0)),
            pl.BlockSpec((1, nhid), lambda i: (0, 0)),
            pl.BlockSpec((nhid, nclass), lambda i: (0, 0)),
        ],
        out_specs=[
            pl.BlockSpec((tm, nclass), lambda i: (i, 0)),
            pl.BlockSpec((1, 3, nfeat), lambda i: (0, 0, 0)),
            pl.BlockSpec((1, 3, nhid), lambda i: (0, 0, 0)),
        ],
        out_shape=[
            jax.ShapeDtypeStruct((n, nclass), jnp.bfloat16),
            jax.ShapeDtypeStruct((1, 3, nfeat), fdt),
            jax.ShapeDtypeStruct((1, 3, nhid), fdt),
        ],
        scratch_shapes=[pltpu.VMEM((n, nhid), jnp.bfloat16)],
        compiler_params=pltpu.CompilerParams(
            dimension_semantics=("arbitrary",)),
    )(adj, x, W1, b1.reshape(1, nhid), W2)

    out, mv = pl.pallas_call(
        _layer2_body,
        grid=(g,),
        in_specs=[
            pl.BlockSpec((tm, n), lambda i: (i, 0)),
            pl.BlockSpec((n, nclass), lambda i: (0, 0)),
            pl.BlockSpec((1, nclass), lambda i: (0, 0)),
            pl.BlockSpec((1, 3, nfeat), lambda i: (0, 0, 0)),
            pl.BlockSpec((1, 3, nhid), lambda i: (0, 0, 0)),
        ],
        out_specs=[
            pl.BlockSpec((tm, nclass), lambda i: (i, 0)),
            pl.BlockSpec((12,), lambda i: (0,)),
        ],
        out_shape=[
            jax.ShapeDtypeStruct((n, nclass), fdt),
            jax.ShapeDtypeStruct((12,), fdt),
        ],
        scratch_shapes=[pltpu.VMEM((3, nclass), fdt)],
        compiler_params=pltpu.CompilerParams(
            dimension_semantics=("arbitrary",)),
    )(adj, v, b2.reshape(1, nclass), xst, hst)

    return (out, mv)
